# baseline (speedup 1.0000x reference)
import functools

import jax
import jax.numpy as jnp
from jax import lax
from jax.experimental import pallas as pl
from jax.experimental.pallas import tpu as pltpu
from jax.experimental.pallas import tpu_sc as plsc

N = 10000
NP = 10240
E = 320000
D_H = 256
D_OUT = 128
BM = 256

_NS = 16
_CH = 40
_RPT = NP // _NS


def _make_agg(split_edges):
    dh = 128
    ept = (E // 2 if split_edges else E) // _NS
    nchunk = ept // _CH
    U = 8
    nbody = nchunk // U
    rem = nchunk - nbody * U
    mesh = plsc.VectorSubcoreMesh(core_axis_name="c", subcore_axis_name="s")

    scratch = ([pltpu.VMEM((_CH,), jnp.int32)] * U
               + [pltpu.VMEM((_CH,), jnp.int32)] * U
               + [pltpu.VMEM((_CH, dh), jnp.float32)] * U
               + [pltpu.VMEM_SHARED((NP, dh), jnp.float32)]
               + [pltpu.SemaphoreType.DMA] * (2 * U + 1))

    @functools.partial(
        pl.kernel,
        out_type=jax.ShapeDtypeStruct((2 * NP, dh), jnp.float32),
        mesh=mesh,
        scratch_types=scratch,
    )
    def agg(h_hbm, src_hbm, dst_hbm, zeros_hbm, out_hbm, *refs):
        sidx = refs[0:U]
        didx = refs[U:2 * U]
        rows = refs[2 * U:3 * U]
        acc = refs[3 * U]
        gsem = refs[3 * U + 1:4 * U + 1]
        isem = refs[4 * U + 1:5 * U + 1]
        ssem = refs[5 * U + 1]

        c = lax.axis_index("c")
        s = lax.axis_index("s")
        zero_cp = pltpu.async_copy(zeros_hbm.at[pl.ds(s * _RPT, _RPT)],
                                   acc.at[pl.ds(s * _RPT, _RPT)], ssem)
        if split_edges:
            src_base = c * (E // 2) + s * ept
            dst_base = c * (E // 2) + s * ept
        else:
            src_base = c * E + s * ept
            dst_base = s * ept

        def fire_idx(p, k):
            off = (p * U + k) * _CH
            pltpu.async_copy(src_hbm.at[pl.ds(src_base + off, _CH)],
                             sidx[k], isem[k])
            pltpu.async_copy(dst_hbm.at[pl.ds(dst_base + off, _CH)],
                             didx[k], isem[k])

        def wait_idx(p, k):
            off = (p * U + k) * _CH
            pltpu.make_async_copy(src_hbm.at[pl.ds(src_base + off, _CH)],
                                  sidx[k], isem[k]).wait()
            pltpu.make_async_copy(dst_hbm.at[pl.ds(dst_base + off, _CH)],
                                  didx[k], isem[k]).wait()

        for k in range(U):
            fire_idx(0, k)
        zero_cp.wait()
        plsc.subcore_barrier()

        def body(p, _):
            gathers = []
            for k in range(U):
                wait_idx(p, k)
                gathers.append(
                    pltpu.async_copy(h_hbm.at[sidx[k]], rows[k], gsem[k]))
            scatters = []
            for k in range(U):
                gathers[k].wait()
                scatters.append(
                    pltpu.async_copy(rows[k], acc.at[didx[k]], ssem, add=True))
            pnext = lax.rem(p + 1, nbody)
            for k in range(U):
                scatters[k].wait()
            for k in range(U):
                fire_idx(pnext, k)
            return ()

        lax.fori_loop(0, nbody, body, (), unroll=False)
        for k in range(U):
            wait_idx(0, k)
        for r in range(rem):
            off = (nbody * U + r) * _CH
            pltpu.sync_copy(src_hbm.at[pl.ds(src_base + off, _CH)], sidx[0])
            pltpu.sync_copy(dst_hbm.at[pl.ds(dst_base + off, _CH)], didx[0])
            pltpu.async_copy(h_hbm.at[sidx[0]], rows[0], gsem[0]).wait()
            pltpu.sync_copy(rows[0], acc.at[didx[0]], add=True)
        plsc.subcore_barrier()
        pltpu.sync_copy(acc.at[pl.ds(s * _RPT, _RPT)],
                        out_hbm.at[pl.ds(c * NP + s * _RPT, _RPT)])

    return agg


_agg_edges = _make_agg(True)
_agg_cols = _make_agg(False)


_NB = NP // BM


def _mlp_stats(t, w1_ref, b1_ref, w2_ref, b2_ref, v_scr, st_scr):
    i = pl.program_id(0)
    u = jax.nn.relu(jnp.dot(t, w1_ref[...],
                            preferred_element_type=jnp.float32) + b1_ref[...])
    v = jnp.dot(u, w2_ref[...], preferred_element_type=jnp.float32) + b2_ref[...]
    v_scr[pl.ds(i * BM, BM), :] = v
    rows = i * BM + lax.broadcasted_iota(jnp.int32, (BM, 1), 0)
    vm = jnp.where(rows < N, v, 0.0)
    s1 = jnp.sum(vm, axis=0, keepdims=True)
    upd = jnp.concatenate(
        [s1, jnp.zeros((7, D_H), jnp.float32)], axis=0)

    @pl.when(i == 0)
    def _():
        st_scr[...] = jnp.zeros_like(st_scr)

    st_scr[...] += upd


def _var_pass(v_scr, st_scr):
    i = pl.program_id(0)
    j = i - _NB
    mu = st_scr[0:1, :] * (1.0 / N)
    v = v_scr[pl.ds(j * BM, BM), :]
    rows = j * BM + lax.broadcasted_iota(jnp.int32, (BM, 1), 0)
    d = jnp.where(rows < N, v - mu, 0.0)
    s2 = jnp.sum(d * d, axis=0, keepdims=True)
    upd = jnp.concatenate(
        [jnp.zeros((1, D_H), jnp.float32), s2,
         jnp.zeros((6, D_H), jnp.float32)], axis=0)
    st_scr[...] += upd


def _bn(v, st, g_ref, be_ref):
    mu = st[0:1, :] * (1.0 / N)
    var = st[1:2, :] * (1.0 / N)
    scale = g_ref[...] * lax.rsqrt(var + 1e-5)
    return jax.nn.relu(scale * (v - mu) + be_ref[...])


def _ab1_body(x_ref, p0_ref, p1_ref, w1_ref, b1_ref, w2_ref, b2_ref,
              g_ref, be_ref, o_ref, v_scr, st_scr):
    i = pl.program_id(0)

    @pl.when(i < _NB)
    def _():
        t = x_ref[...] + p0_ref[0] + p1_ref[0]
        _mlp_stats(t, w1_ref, b1_ref, w2_ref, b2_ref, v_scr, st_scr)

    @pl.when(jnp.logical_and(i >= _NB, i < 2 * _NB))
    def _():
        _var_pass(v_scr, st_scr)

    @pl.when(i >= 2 * _NB)
    def _():
        v = v_scr[pl.ds((i - 2 * _NB) * BM, BM), :]
        res = _bn(v, st_scr[...], g_ref, be_ref)
        o_ref[0] = res[:, :128]
        o_ref[1] = res[:, 128:]


def _ab_body(hl_ref, hh_ref, al_ref, ah_ref, w1_ref, b1_ref, w2_ref, b2_ref,
             g_ref, be_ref, o_ref, v_scr, st_scr):
    i = pl.program_id(0)

    @pl.when(i < _NB)
    def _():
        t = jnp.concatenate([hl_ref[0] + al_ref[0], hh_ref[0] + ah_ref[0]],
                            axis=1)
        _mlp_stats(t, w1_ref, b1_ref, w2_ref, b2_ref, v_scr, st_scr)

    @pl.when(jnp.logical_and(i >= _NB, i < 2 * _NB))
    def _():
        _var_pass(v_scr, st_scr)

    @pl.when(i >= 2 * _NB)
    def _():
        v = v_scr[pl.ds((i - 2 * _NB) * BM, BM), :]
        res = _bn(v, st_scr[...], g_ref, be_ref)
        o_ref[0] = res[:, :128]
        o_ref[1] = res[:, 128:]


def _ab3_body(hl_ref, hh_ref, al_ref, ah_ref, w1_ref, b1_ref, w2_ref, b2_ref,
              g_ref, be_ref, wo_ref, bo_ref, o_ref, v_scr, st_scr):
    i = pl.program_id(0)

    @pl.when(i < _NB)
    def _():
        t = jnp.concatenate([hl_ref[0] + al_ref[0], hh_ref[0] + ah_ref[0]],
                            axis=1)
        _mlp_stats(t, w1_ref, b1_ref, w2_ref, b2_ref, v_scr, st_scr)

    @pl.when(jnp.logical_and(i >= _NB, i < 2 * _NB))
    def _():
        _var_pass(v_scr, st_scr)

    @pl.when(i >= 2 * _NB)
    def _():
        v = v_scr[pl.ds((i - 2 * _NB) * BM, BM), :]
        res = _bn(v, st_scr[...], g_ref, be_ref)
        o_ref[...] = jnp.dot(res, wo_ref[...],
                             preferred_element_type=jnp.float32) + bo_ref[...]


def _blk(i):
    return jnp.minimum(i, _NB - 1)


def _oblk(i):
    return jnp.maximum(i - 2 * _NB, 0)


_SCRATCH = [
    pltpu.VMEM((NP, D_H), jnp.float32),
    pltpu.VMEM((8, D_H), jnp.float32),
]

_W_SPECS = [
    pl.BlockSpec((D_H, D_H), lambda i: (0, 0)),
    pl.BlockSpec((1, D_H), lambda i: (0, 0)),
    pl.BlockSpec((D_H, D_H), lambda i: (0, 0)),
    pl.BlockSpec((1, D_H), lambda i: (0, 0)),
    pl.BlockSpec((1, D_H), lambda i: (0, 0)),
    pl.BlockSpec((1, D_H), lambda i: (0, 0)),
]


def _call_ab1(xp, a3, W1, b1, W2, b2, g, be):
    specs = [
        pl.BlockSpec((BM, 128), lambda i: (_blk(i), 0)),
        pl.BlockSpec((1, BM, 128), lambda i: (0, _blk(i), 0)),
        pl.BlockSpec((1, BM, 128), lambda i: (1, _blk(i), 0)),
        pl.BlockSpec((128, D_H), lambda i: (0, 0)),
    ] + _W_SPECS[1:]
    return pl.pallas_call(
        _ab1_body,
        grid=(3 * _NB,),
        in_specs=specs,
        out_specs=pl.BlockSpec((2, BM, 128), lambda i: (0, _oblk(i), 0)),
        out_shape=jax.ShapeDtypeStruct((2, NP, 128), jnp.float32),
        scratch_shapes=_SCRATCH,
    )(xp, a3, a3, W1, b1.reshape(1, D_H), W2, b2.reshape(1, D_H),
      g.reshape(1, D_H), be.reshape(1, D_H))


_H_SPECS = [
    pl.BlockSpec((1, BM, 128), lambda i: (0, _blk(i), 0)),
    pl.BlockSpec((1, BM, 128), lambda i: (1, _blk(i), 0)),
    pl.BlockSpec((1, BM, 128), lambda i: (0, _blk(i), 0)),
    pl.BlockSpec((1, BM, 128), lambda i: (1, _blk(i), 0)),
]


def _call_ab(h3, a3, W1, b1, W2, b2, g, be):
    return pl.pallas_call(
        _ab_body,
        grid=(3 * _NB,),
        in_specs=_H_SPECS + _W_SPECS,
        out_specs=pl.BlockSpec((2, BM, 128), lambda i: (0, _oblk(i), 0)),
        out_shape=jax.ShapeDtypeStruct((2, NP, 128), jnp.float32),
        scratch_shapes=_SCRATCH,
    )(h3, h3, a3, a3, W1, b1.reshape(1, D_H), W2, b2.reshape(1, D_H),
      g.reshape(1, D_H), be.reshape(1, D_H))


def _call_ab3(h3, a3, W1, b1, W2, b2, g, be, Wo, bo):
    specs = _H_SPECS + _W_SPECS + [
        pl.BlockSpec((D_H, D_OUT), lambda i: (0, 0)),
        pl.BlockSpec((1, D_OUT), lambda i: (0, 0)),
    ]
    return pl.pallas_call(
        _ab3_body,
        grid=(3 * _NB,),
        in_specs=specs,
        out_specs=pl.BlockSpec((BM, D_OUT), lambda i: (_oblk(i), 0)),
        out_shape=jax.ShapeDtypeStruct((NP, D_OUT), jnp.float32),
        scratch_shapes=_SCRATCH,
    )(h3, h3, a3, a3, W1, b1.reshape(1, D_H), W2, b2.reshape(1, D_H),
      g.reshape(1, D_H), be.reshape(1, D_H), Wo, bo.reshape(1, D_OUT))


def kernel(x, edge_index, W1_0, b1_0, W2_0, b2_0, g_0, be_0, W1_1, b1_1,
           W2_1, b2_1, g_1, be_1, W1_2, b1_2, W2_2, b2_2, g_2, be_2, Wo, bo):
    z128 = jnp.zeros((NP, 128), jnp.float32)
    src = edge_index[0]
    dst = edge_index[1]
    src2 = jnp.concatenate([src, src + NP])

    xp = jnp.pad(x, ((0, NP - N), (0, 0)))

    a1 = _agg_edges(xp, src, dst, z128)
    h3 = _call_ab1(xp, a1.reshape(2, NP, 128), W1_0, b1_0, W2_0, b2_0,
                   g_0, be_0)

    a2 = _agg_cols(h3.reshape(2 * NP, 128), src2, dst, z128)
    h3 = _call_ab(h3, a2.reshape(2, NP, 128), W1_1, b1_1, W2_1, b2_1,
                  g_1, be_1)

    a3 = _agg_cols(h3.reshape(2 * NP, 128), src2, dst, z128)
    out = _call_ab3(h3, a3.reshape(2, NP, 128), W1_2, b1_2, W2_2, b2_2,
                    g_2, be_2, Wo, bo)
    return out[:N]


# speedup vs baseline: 1.0649x; 1.0649x over previous
import functools

import jax
import jax.numpy as jnp
from jax import lax
from jax.experimental import pallas as pl
from jax.experimental.pallas import tpu as pltpu
from jax.experimental.pallas import tpu_sc as plsc

N = 10000
NP = 10240
E = 320000
D_H = 256
D_OUT = 128
BM = 256

_NS = 16
_CH = 40
_NACC = 10112
_RPT = _NACC // _NS


def _make_agg(split_edges):
    dh = 128
    ept = (E // 2 if split_edges else E) // _NS
    nchunk = ept // _CH
    U = 9
    nbody = nchunk // U
    rem = nchunk - nbody * U
    mesh = plsc.VectorSubcoreMesh(core_axis_name="c", subcore_axis_name="s")

    scratch = ([pltpu.VMEM((_CH,), jnp.int32)] * U
               + [pltpu.VMEM((_CH,), jnp.int32)] * U
               + [pltpu.VMEM((_CH, dh), jnp.float32)] * U
               + [pltpu.VMEM_SHARED((_NACC, dh), jnp.float32)]
               + [pltpu.SemaphoreType.DMA] * (2 * U + 1))

    @functools.partial(
        pl.kernel,
        out_type=jax.ShapeDtypeStruct((2 * NP, dh), jnp.float32),
        mesh=mesh,
        scratch_types=scratch,
    )
    def agg(h_hbm, src_hbm, dst_hbm, zeros_hbm, out_hbm, *refs):
        sidx = refs[0:U]
        didx = refs[U:2 * U]
        rows = refs[2 * U:3 * U]
        acc = refs[3 * U]
        gsem = refs[3 * U + 1:4 * U + 1]
        isem = refs[4 * U + 1:5 * U + 1]
        ssem = refs[5 * U + 1]

        c = lax.axis_index("c")
        s = lax.axis_index("s")
        zero_cp = pltpu.async_copy(zeros_hbm.at[pl.ds(s * _RPT, _RPT)],
                                   acc.at[pl.ds(s * _RPT, _RPT)], ssem)
        if split_edges:
            src_base = c * (E // 2) + s * ept
            dst_base = c * (E // 2) + s * ept
        else:
            src_base = c * E + s * ept
            dst_base = s * ept

        def fire_idx(p, k):
            off = (p * U + k) * _CH
            pltpu.async_copy(src_hbm.at[pl.ds(src_base + off, _CH)],
                             sidx[k], isem[k])
            pltpu.async_copy(dst_hbm.at[pl.ds(dst_base + off, _CH)],
                             didx[k], isem[k])

        def wait_idx(p, k):
            off = (p * U + k) * _CH
            pltpu.make_async_copy(src_hbm.at[pl.ds(src_base + off, _CH)],
                                  sidx[k], isem[k]).wait()
            pltpu.make_async_copy(dst_hbm.at[pl.ds(dst_base + off, _CH)],
                                  didx[k], isem[k]).wait()

        for k in range(U):
            fire_idx(0, k)
        zero_cp.wait()
        plsc.subcore_barrier()

        def body(p, _):
            gathers = []
            for k in range(U):
                wait_idx(p, k)
                gathers.append(
                    pltpu.async_copy(h_hbm.at[sidx[k]], rows[k], gsem[k]))
            scatters = []
            for k in range(U):
                gathers[k].wait()
                scatters.append(
                    pltpu.async_copy(rows[k], acc.at[didx[k]], ssem, add=True))
            pnext = lax.rem(p + 1, nbody)
            for k in range(U):
                scatters[k].wait()
            for k in range(U):
                fire_idx(pnext, k)
            return ()

        lax.fori_loop(0, nbody, body, (), unroll=False)
        for k in range(U):
            wait_idx(0, k)
        for r in range(rem):
            off = (nbody * U + r) * _CH
            pltpu.sync_copy(src_hbm.at[pl.ds(src_base + off, _CH)], sidx[0])
            pltpu.sync_copy(dst_hbm.at[pl.ds(dst_base + off, _CH)], didx[0])
            pltpu.async_copy(h_hbm.at[sidx[0]], rows[0], gsem[0]).wait()
            pltpu.sync_copy(rows[0], acc.at[didx[0]], add=True)
        plsc.subcore_barrier()
        pltpu.sync_copy(acc.at[pl.ds(s * _RPT, _RPT)],
                        out_hbm.at[pl.ds(c * NP + s * _RPT, _RPT)])

    return agg


_agg_edges = _make_agg(True)
_agg_cols = _make_agg(False)


_NB = NP // BM


def _mlp_stats(t, w1_ref, b1_ref, w2_ref, b2_ref, v_scr, st_scr):
    i = pl.program_id(0)
    u = jax.nn.relu(jnp.dot(t, w1_ref[...],
                            preferred_element_type=jnp.float32) + b1_ref[...])
    v = jnp.dot(u, w2_ref[...], preferred_element_type=jnp.float32) + b2_ref[...]
    v_scr[pl.ds(i * BM, BM), :] = v
    rows = i * BM + lax.broadcasted_iota(jnp.int32, (BM, 1), 0)
    vm = jnp.where(rows < N, v, 0.0)
    s1 = jnp.sum(vm, axis=0, keepdims=True)
    s2 = jnp.sum(vm * vm, axis=0, keepdims=True)
    upd = jnp.concatenate(
        [s1, s2, jnp.zeros((6, D_H), jnp.float32)], axis=0)

    @pl.when(i == 0)
    def _():
        st_scr[...] = jnp.zeros_like(st_scr)

    st_scr[...] += upd


def _bn(v, st, g_ref, be_ref):
    mu = st[0:1, :] * (1.0 / N)
    ex2 = st[1:2, :] * (1.0 / N)
    var = ex2 - mu * mu
    scale = g_ref[...] * lax.rsqrt(var + 1e-5)
    return jax.nn.relu(scale * (v - mu) + be_ref[...])


def _ab1_body(x_ref, p0_ref, p1_ref, w1_ref, b1_ref, w2_ref, b2_ref,
              g_ref, be_ref, o_ref, v_scr, st_scr):
    i = pl.program_id(0)

    @pl.when(i < _NB)
    def _():
        t = x_ref[...] + p0_ref[0] + p1_ref[0]
        _mlp_stats(t, w1_ref, b1_ref, w2_ref, b2_ref, v_scr, st_scr)

    @pl.when(i >= _NB)
    def _():
        v = v_scr[pl.ds((i - _NB) * BM, BM), :]
        res = _bn(v, st_scr[...], g_ref, be_ref)
        o_ref[0] = res[:, :128]
        o_ref[1] = res[:, 128:]


def _ab_body(hl_ref, hh_ref, al_ref, ah_ref, w1_ref, b1_ref, w2_ref, b2_ref,
             g_ref, be_ref, o_ref, v_scr, st_scr):
    i = pl.program_id(0)

    @pl.when(i < _NB)
    def _():
        t = jnp.concatenate([hl_ref[0] + al_ref[0], hh_ref[0] + ah_ref[0]],
                            axis=1)
        _mlp_stats(t, w1_ref, b1_ref, w2_ref, b2_ref, v_scr, st_scr)

    @pl.when(i >= _NB)
    def _():
        v = v_scr[pl.ds((i - _NB) * BM, BM), :]
        res = _bn(v, st_scr[...], g_ref, be_ref)
        o_ref[0] = res[:, :128]
        o_ref[1] = res[:, 128:]


def _ab3_body(hl_ref, hh_ref, al_ref, ah_ref, w1_ref, b1_ref, w2_ref, b2_ref,
              g_ref, be_ref, wo_ref, bo_ref, o_ref, v_scr, st_scr):
    i = pl.program_id(0)

    @pl.when(i < _NB)
    def _():
        t = jnp.concatenate([hl_ref[0] + al_ref[0], hh_ref[0] + ah_ref[0]],
                            axis=1)
        _mlp_stats(t, w1_ref, b1_ref, w2_ref, b2_ref, v_scr, st_scr)

    @pl.when(i >= _NB)
    def _():
        v = v_scr[pl.ds((i - _NB) * BM, BM), :]
        res = _bn(v, st_scr[...], g_ref, be_ref)
        o_ref[...] = jnp.dot(res, wo_ref[...],
                             preferred_element_type=jnp.float32) + bo_ref[...]


def _blk(i):
    return jnp.minimum(i, _NB - 1)


def _oblk(i):
    return jnp.maximum(i - _NB, 0)


_SCRATCH = [
    pltpu.VMEM((NP, D_H), jnp.float32),
    pltpu.VMEM((8, D_H), jnp.float32),
]

_W_SPECS = [
    pl.BlockSpec((D_H, D_H), lambda i: (0, 0)),
    pl.BlockSpec((1, D_H), lambda i: (0, 0)),
    pl.BlockSpec((D_H, D_H), lambda i: (0, 0)),
    pl.BlockSpec((1, D_H), lambda i: (0, 0)),
    pl.BlockSpec((1, D_H), lambda i: (0, 0)),
    pl.BlockSpec((1, D_H), lambda i: (0, 0)),
]


def _call_ab1(xp, a3, W1, b1, W2, b2, g, be):
    specs = [
        pl.BlockSpec((BM, 128), lambda i: (_blk(i), 0)),
        pl.BlockSpec((1, BM, 128), lambda i: (0, _blk(i), 0)),
        pl.BlockSpec((1, BM, 128), lambda i: (1, _blk(i), 0)),
        pl.BlockSpec((128, D_H), lambda i: (0, 0)),
    ] + _W_SPECS[1:]
    return pl.pallas_call(
        _ab1_body,
        grid=(2 * _NB,),
        in_specs=specs,
        out_specs=pl.BlockSpec((2, BM, 128), lambda i: (0, _oblk(i), 0)),
        out_shape=jax.ShapeDtypeStruct((2, NP, 128), jnp.float32),
        scratch_shapes=_SCRATCH,
    )(xp, a3, a3, W1, b1.reshape(1, D_H), W2, b2.reshape(1, D_H),
      g.reshape(1, D_H), be.reshape(1, D_H))


_H_SPECS = [
    pl.BlockSpec((1, BM, 128), lambda i: (0, _blk(i), 0)),
    pl.BlockSpec((1, BM, 128), lambda i: (1, _blk(i), 0)),
    pl.BlockSpec((1, BM, 128), lambda i: (0, _blk(i), 0)),
    pl.BlockSpec((1, BM, 128), lambda i: (1, _blk(i), 0)),
]


def _call_ab(h3, a3, W1, b1, W2, b2, g, be):
    return pl.pallas_call(
        _ab_body,
        grid=(2 * _NB,),
        in_specs=_H_SPECS + _W_SPECS,
        out_specs=pl.BlockSpec((2, BM, 128), lambda i: (0, _oblk(i), 0)),
        out_shape=jax.ShapeDtypeStruct((2, NP, 128), jnp.float32),
        scratch_shapes=_SCRATCH,
    )(h3, h3, a3, a3, W1, b1.reshape(1, D_H), W2, b2.reshape(1, D_H),
      g.reshape(1, D_H), be.reshape(1, D_H))


def _call_ab3(h3, a3, W1, b1, W2, b2, g, be, Wo, bo):
    specs = _H_SPECS + _W_SPECS + [
        pl.BlockSpec((D_H, D_OUT), lambda i: (0, 0)),
        pl.BlockSpec((1, D_OUT), lambda i: (0, 0)),
    ]
    return pl.pallas_call(
        _ab3_body,
        grid=(2 * _NB,),
        in_specs=specs,
        out_specs=pl.BlockSpec((BM, D_OUT), lambda i: (_oblk(i), 0)),
        out_shape=jax.ShapeDtypeStruct((NP, D_OUT), jnp.float32),
        scratch_shapes=_SCRATCH,
    )(h3, h3, a3, a3, W1, b1.reshape(1, D_H), W2, b2.reshape(1, D_H),
      g.reshape(1, D_H), be.reshape(1, D_H), Wo, bo.reshape(1, D_OUT))


def kernel(x, edge_index, W1_0, b1_0, W2_0, b2_0, g_0, be_0, W1_1, b1_1,
           W2_1, b2_1, g_1, be_1, W1_2, b1_2, W2_2, b2_2, g_2, be_2, Wo, bo):
    z128 = jnp.zeros((NP, 128), jnp.float32)
    src = edge_index[0]
    dst = edge_index[1]
    src2 = jnp.concatenate([src, src + NP])

    xp = jnp.pad(x, ((0, NP - N), (0, 0)))

    a1 = _agg_edges(xp, src, dst, z128)
    h3 = _call_ab1(xp, a1.reshape(2, NP, 128), W1_0, b1_0, W2_0, b2_0,
                   g_0, be_0)

    a2 = _agg_cols(h3.reshape(2 * NP, 128), src2, dst, z128)
    h3 = _call_ab(h3, a2.reshape(2, NP, 128), W1_1, b1_1, W2_1, b2_1,
                  g_1, be_1)

    a3 = _agg_cols(h3.reshape(2 * NP, 128), src2, dst, z128)
    out = _call_ab3(h3, a3.reshape(2, NP, 128), W1_2, b1_2, W2_2, b2_2,
                    g_2, be_2, Wo, bo)
    return out[:N]


# speedup vs baseline: 1.1389x; 1.0695x over previous
import functools

import jax
import jax.numpy as jnp
from jax import lax
from jax.experimental import pallas as pl
from jax.experimental.pallas import tpu as pltpu
from jax.experimental.pallas import tpu_sc as plsc

N = 10000
NP = 10240
E = 320000
D_H = 256
D_OUT = 128
BM = 512

_NS = 16
_CH = 40
_NACC = 10112
_RPT = _NACC // _NS


def _make_agg(split_edges):
    dh = 128
    ept = (E // 2 if split_edges else E) // _NS
    nchunk = ept // _CH
    U = 9
    nbody = nchunk // U
    rem = nchunk - nbody * U
    mesh = plsc.VectorSubcoreMesh(core_axis_name="c", subcore_axis_name="s")

    scratch = ([pltpu.VMEM((_CH,), jnp.int32)] * U
               + [pltpu.VMEM((_CH,), jnp.int32)] * U
               + [pltpu.VMEM((_CH, dh), jnp.float32)] * U
               + [pltpu.VMEM_SHARED((_NACC, dh), jnp.float32)]
               + [pltpu.SemaphoreType.DMA] * (2 * U + 1))

    @functools.partial(
        pl.kernel,
        out_type=jax.ShapeDtypeStruct((2 * NP, dh), jnp.float32),
        mesh=mesh,
        scratch_types=scratch,
    )
    def agg(h_hbm, src_hbm, dst_hbm, zeros_hbm, out_hbm, *refs):
        sidx = refs[0:U]
        didx = refs[U:2 * U]
        rows = refs[2 * U:3 * U]
        acc = refs[3 * U]
        gsem = refs[3 * U + 1:4 * U + 1]
        isem = refs[4 * U + 1:5 * U + 1]
        ssem = refs[5 * U + 1]

        c = lax.axis_index("c")
        s = lax.axis_index("s")
        zero_cp = pltpu.async_copy(zeros_hbm.at[pl.ds(s * _RPT, _RPT)],
                                   acc.at[pl.ds(s * _RPT, _RPT)], ssem)
        if split_edges:
            src_base = c * (E // 2) + s * ept
            dst_base = c * (E // 2) + s * ept
        else:
            src_base = c * E + s * ept
            dst_base = s * ept

        def fire_idx(p, k):
            off = (p * U + k) * _CH
            pltpu.async_copy(src_hbm.at[pl.ds(src_base + off, _CH)],
                             sidx[k], isem[k])
            pltpu.async_copy(dst_hbm.at[pl.ds(dst_base + off, _CH)],
                             didx[k], isem[k])

        def wait_idx(p, k):
            off = (p * U + k) * _CH
            pltpu.make_async_copy(src_hbm.at[pl.ds(src_base + off, _CH)],
                                  sidx[k], isem[k]).wait()
            pltpu.make_async_copy(dst_hbm.at[pl.ds(dst_base + off, _CH)],
                                  didx[k], isem[k]).wait()

        for k in range(U):
            fire_idx(0, k)
        zero_cp.wait()
        plsc.subcore_barrier()

        def body(p, _):
            gathers = []
            for k in range(U):
                wait_idx(p, k)
                gathers.append(
                    pltpu.async_copy(h_hbm.at[sidx[k]], rows[k], gsem[k]))
            scatters = []
            for k in range(U):
                gathers[k].wait()
                scatters.append(
                    pltpu.async_copy(rows[k], acc.at[didx[k]], ssem, add=True))
            pnext = lax.rem(p + 1, nbody)
            for k in range(U):
                scatters[k].wait()
            for k in range(U):
                fire_idx(pnext, k)
            return ()

        lax.fori_loop(0, nbody, body, (), unroll=False)
        for k in range(U):
            wait_idx(0, k)
        for r in range(rem):
            off = (nbody * U + r) * _CH
            pltpu.sync_copy(src_hbm.at[pl.ds(src_base + off, _CH)], sidx[0])
            pltpu.sync_copy(dst_hbm.at[pl.ds(dst_base + off, _CH)], didx[0])
            pltpu.async_copy(h_hbm.at[sidx[0]], rows[0], gsem[0]).wait()
            pltpu.sync_copy(rows[0], acc.at[didx[0]], add=True)
        plsc.subcore_barrier()
        pltpu.sync_copy(acc.at[pl.ds(s * _RPT, _RPT)],
                        out_hbm.at[pl.ds(c * NP + s * _RPT, _RPT)])

    return agg


_agg_edges = _make_agg(True)
_agg_cols = _make_agg(False)


_NB = NP // BM


def _mlp_stats(t, w1_ref, b1_ref, w2_ref, b2_ref, v_scr, st_scr):
    i = pl.program_id(0)
    u = jax.nn.relu(jnp.dot(t, w1_ref[...],
                            preferred_element_type=jnp.float32) + b1_ref[...])
    v = jnp.dot(u, w2_ref[...], preferred_element_type=jnp.float32) + b2_ref[...]
    v_scr[pl.ds(i * BM, BM), :] = v
    rows = i * BM + lax.broadcasted_iota(jnp.int32, (BM, 1), 0)
    vm = jnp.where(rows < N, v, 0.0)
    s1 = jnp.sum(vm, axis=0, keepdims=True)
    s2 = jnp.sum(vm * vm, axis=0, keepdims=True)
    upd = jnp.concatenate(
        [s1, s2, jnp.zeros((6, D_H), jnp.float32)], axis=0)

    @pl.when(i == 0)
    def _():
        st_scr[...] = jnp.zeros_like(st_scr)

    st_scr[...] += upd


def _bn(v, st, g_ref, be_ref):
    mu = st[0:1, :] * (1.0 / N)
    ex2 = st[1:2, :] * (1.0 / N)
    var = ex2 - mu * mu
    scale = g_ref[...] * lax.rsqrt(var + 1e-5)
    return jax.nn.relu(scale * (v - mu) + be_ref[...])


def _ab1_body(x_ref, p0_ref, p1_ref, w1_ref, b1_ref, w2_ref, b2_ref,
              g_ref, be_ref, o_ref, v_scr, st_scr):
    i = pl.program_id(0)

    @pl.when(i < _NB)
    def _():
        t = x_ref[...] + p0_ref[0] + p1_ref[0]
        _mlp_stats(t, w1_ref, b1_ref, w2_ref, b2_ref, v_scr, st_scr)

    @pl.when(i >= _NB)
    def _():
        v = v_scr[pl.ds((i - _NB) * BM, BM), :]
        res = _bn(v, st_scr[...], g_ref, be_ref)
        o_ref[0] = res[:, :128]
        o_ref[1] = res[:, 128:]


def _ab_body(hl_ref, hh_ref, al_ref, ah_ref, w1_ref, b1_ref, w2_ref, b2_ref,
             g_ref, be_ref, o_ref, v_scr, st_scr):
    i = pl.program_id(0)

    @pl.when(i < _NB)
    def _():
        t = jnp.concatenate([hl_ref[0] + al_ref[0], hh_ref[0] + ah_ref[0]],
                            axis=1)
        _mlp_stats(t, w1_ref, b1_ref, w2_ref, b2_ref, v_scr, st_scr)

    @pl.when(i >= _NB)
    def _():
        v = v_scr[pl.ds((i - _NB) * BM, BM), :]
        res = _bn(v, st_scr[...], g_ref, be_ref)
        o_ref[0] = res[:, :128]
        o_ref[1] = res[:, 128:]


def _ab3_body(hl_ref, hh_ref, al_ref, ah_ref, w1_ref, b1_ref, w2_ref, b2_ref,
              g_ref, be_ref, wo_ref, bo_ref, o_ref, v_scr, st_scr):
    i = pl.program_id(0)

    @pl.when(i < _NB)
    def _():
        t = jnp.concatenate([hl_ref[0] + al_ref[0], hh_ref[0] + ah_ref[0]],
                            axis=1)
        _mlp_stats(t, w1_ref, b1_ref, w2_ref, b2_ref, v_scr, st_scr)

    @pl.when(i >= _NB)
    def _():
        v = v_scr[pl.ds((i - _NB) * BM, BM), :]
        res = _bn(v, st_scr[...], g_ref, be_ref)
        o_ref[...] = jnp.dot(res, wo_ref[...],
                             preferred_element_type=jnp.float32) + bo_ref[...]


def _blk(i):
    return jnp.minimum(i, _NB - 1)


def _oblk(i):
    return jnp.maximum(i - _NB, 0)


_SCRATCH = [
    pltpu.VMEM((NP, D_H), jnp.float32),
    pltpu.VMEM((8, D_H), jnp.float32),
]

_W_SPECS = [
    pl.BlockSpec((D_H, D_H), lambda i: (0, 0)),
    pl.BlockSpec((1, D_H), lambda i: (0, 0)),
    pl.BlockSpec((D_H, D_H), lambda i: (0, 0)),
    pl.BlockSpec((1, D_H), lambda i: (0, 0)),
    pl.BlockSpec((1, D_H), lambda i: (0, 0)),
    pl.BlockSpec((1, D_H), lambda i: (0, 0)),
]


def _call_ab1(xp, a3, W1, b1, W2, b2, g, be):
    specs = [
        pl.BlockSpec((BM, 128), lambda i: (_blk(i), 0)),
        pl.BlockSpec((1, BM, 128), lambda i: (0, _blk(i), 0)),
        pl.BlockSpec((1, BM, 128), lambda i: (1, _blk(i), 0)),
        pl.BlockSpec((128, D_H), lambda i: (0, 0)),
    ] + _W_SPECS[1:]
    return pl.pallas_call(
        _ab1_body,
        grid=(2 * _NB,),
        in_specs=specs,
        out_specs=pl.BlockSpec((2, BM, 128), lambda i: (0, _oblk(i), 0)),
        out_shape=jax.ShapeDtypeStruct((2, NP, 128), jnp.float32),
        scratch_shapes=_SCRATCH,
    )(xp, a3, a3, W1, b1.reshape(1, D_H), W2, b2.reshape(1, D_H),
      g.reshape(1, D_H), be.reshape(1, D_H))


_H_SPECS = [
    pl.BlockSpec((1, BM, 128), lambda i: (0, _blk(i), 0)),
    pl.BlockSpec((1, BM, 128), lambda i: (1, _blk(i), 0)),
    pl.BlockSpec((1, BM, 128), lambda i: (0, _blk(i), 0)),
    pl.BlockSpec((1, BM, 128), lambda i: (1, _blk(i), 0)),
]


def _call_ab(h3, a3, W1, b1, W2, b2, g, be):
    return pl.pallas_call(
        _ab_body,
        grid=(2 * _NB,),
        in_specs=_H_SPECS + _W_SPECS,
        out_specs=pl.BlockSpec((2, BM, 128), lambda i: (0, _oblk(i), 0)),
        out_shape=jax.ShapeDtypeStruct((2, NP, 128), jnp.float32),
        scratch_shapes=_SCRATCH,
    )(h3, h3, a3, a3, W1, b1.reshape(1, D_H), W2, b2.reshape(1, D_H),
      g.reshape(1, D_H), be.reshape(1, D_H))


def _call_ab3(h3, a3, W1, b1, W2, b2, g, be, Wo, bo):
    specs = _H_SPECS + _W_SPECS + [
        pl.BlockSpec((D_H, D_OUT), lambda i: (0, 0)),
        pl.BlockSpec((1, D_OUT), lambda i: (0, 0)),
    ]
    return pl.pallas_call(
        _ab3_body,
        grid=(2 * _NB,),
        in_specs=specs,
        out_specs=pl.BlockSpec((BM, D_OUT), lambda i: (_oblk(i), 0)),
        out_shape=jax.ShapeDtypeStruct((NP, D_OUT), jnp.float32),
        scratch_shapes=_SCRATCH,
    )(h3, h3, a3, a3, W1, b1.reshape(1, D_H), W2, b2.reshape(1, D_H),
      g.reshape(1, D_H), be.reshape(1, D_H), Wo, bo.reshape(1, D_OUT))


def kernel(x, edge_index, W1_0, b1_0, W2_0, b2_0, g_0, be_0, W1_1, b1_1,
           W2_1, b2_1, g_1, be_1, W1_2, b1_2, W2_2, b2_2, g_2, be_2, Wo, bo):
    z128 = jnp.zeros((NP, 128), jnp.float32)
    src = edge_index[0]
    dst = edge_index[1]
    src2 = jnp.concatenate([src, src + NP])

    xp = jnp.pad(x, ((0, NP - N), (0, 0)))

    a1 = _agg_edges(xp, src, dst, z128)
    h3 = _call_ab1(xp, a1.reshape(2, NP, 128), W1_0, b1_0, W2_0, b2_0,
                   g_0, be_0)

    a2 = _agg_cols(h3.reshape(2 * NP, 128), src2, dst, z128)
    h3 = _call_ab(h3, a2.reshape(2, NP, 128), W1_1, b1_1, W2_1, b2_1,
                  g_1, be_1)

    a3 = _agg_cols(h3.reshape(2 * NP, 128), src2, dst, z128)
    out = _call_ab3(h3, a3.reshape(2, NP, 128), W1_2, b1_2, W2_2, b2_2,
                    g_2, be_2, Wo, bo)
    return out[:N]


# speedup vs baseline: 1.1799x; 1.0360x over previous
import functools

import jax
import jax.numpy as jnp
from jax import lax
from jax.experimental import pallas as pl
from jax.experimental.pallas import tpu as pltpu
from jax.experimental.pallas import tpu_sc as plsc

N = 10000
NP = 10240
E = 320000
D_H = 256
D_OUT = 128
BM = 1024

_NS = 16
_CH = 40
_NACC = 10112
_RPT = _NACC // _NS


def _make_agg(split_edges):
    dh = 128
    ept = (E // 2 if split_edges else E) // _NS
    nchunk = ept // _CH
    U = 9
    nbody = nchunk // U
    rem = nchunk - nbody * U
    mesh = plsc.VectorSubcoreMesh(core_axis_name="c", subcore_axis_name="s")

    scratch = ([pltpu.VMEM((_CH,), jnp.int32)] * U
               + [pltpu.VMEM((_CH,), jnp.int32)] * U
               + [pltpu.VMEM((_CH, dh), jnp.float32)] * U
               + [pltpu.VMEM_SHARED((_NACC, dh), jnp.float32)]
               + [pltpu.SemaphoreType.DMA] * (2 * U + 1))

    @functools.partial(
        pl.kernel,
        out_type=jax.ShapeDtypeStruct((2 * NP, dh), jnp.float32),
        mesh=mesh,
        scratch_types=scratch,
    )
    def agg(h_hbm, src_hbm, dst_hbm, zeros_hbm, out_hbm, *refs):
        sidx = refs[0:U]
        didx = refs[U:2 * U]
        rows = refs[2 * U:3 * U]
        acc = refs[3 * U]
        gsem = refs[3 * U + 1:4 * U + 1]
        isem = refs[4 * U + 1:5 * U + 1]
        ssem = refs[5 * U + 1]

        c = lax.axis_index("c")
        s = lax.axis_index("s")
        zero_cp = pltpu.async_copy(zeros_hbm.at[pl.ds(s * _RPT, _RPT)],
                                   acc.at[pl.ds(s * _RPT, _RPT)], ssem)
        if split_edges:
            src_base = c * (E // 2) + s * ept
            dst_base = c * (E // 2) + s * ept
        else:
            src_base = c * E + s * ept
            dst_base = s * ept

        def fire_idx(p, k):
            off = (p * U + k) * _CH
            pltpu.async_copy(src_hbm.at[pl.ds(src_base + off, _CH)],
                             sidx[k], isem[k])
            pltpu.async_copy(dst_hbm.at[pl.ds(dst_base + off, _CH)],
                             didx[k], isem[k])

        def wait_idx(p, k):
            off = (p * U + k) * _CH
            pltpu.make_async_copy(src_hbm.at[pl.ds(src_base + off, _CH)],
                                  sidx[k], isem[k]).wait()
            pltpu.make_async_copy(dst_hbm.at[pl.ds(dst_base + off, _CH)],
                                  didx[k], isem[k]).wait()

        for k in range(U):
            fire_idx(0, k)
        zero_cp.wait()
        plsc.subcore_barrier()

        def body(p, _):
            gathers = []
            for k in range(U):
                wait_idx(p, k)
                gathers.append(
                    pltpu.async_copy(h_hbm.at[sidx[k]], rows[k], gsem[k]))
            scatters = []
            for k in range(U):
                gathers[k].wait()
                scatters.append(
                    pltpu.async_copy(rows[k], acc.at[didx[k]], ssem, add=True))
            pnext = lax.rem(p + 1, nbody)
            for k in range(U):
                scatters[k].wait()
            for k in range(U):
                fire_idx(pnext, k)
            return ()

        lax.fori_loop(0, nbody, body, (), unroll=False)
        for k in range(U):
            wait_idx(0, k)
        for r in range(rem):
            off = (nbody * U + r) * _CH
            pltpu.sync_copy(src_hbm.at[pl.ds(src_base + off, _CH)], sidx[0])
            pltpu.sync_copy(dst_hbm.at[pl.ds(dst_base + off, _CH)], didx[0])
            pltpu.async_copy(h_hbm.at[sidx[0]], rows[0], gsem[0]).wait()
            pltpu.sync_copy(rows[0], acc.at[didx[0]], add=True)
        plsc.subcore_barrier()
        pltpu.sync_copy(acc.at[pl.ds(s * _RPT, _RPT)],
                        out_hbm.at[pl.ds(c * NP + s * _RPT, _RPT)])

    return agg


_agg_edges = _make_agg(True)
_agg_cols = _make_agg(False)


_NB = NP // BM


def _mlp_stats(t, w1_ref, b1_ref, w2_ref, b2_ref, v_scr, st_scr):
    i = pl.program_id(0)
    u = jax.nn.relu(jnp.dot(t, w1_ref[...],
                            preferred_element_type=jnp.float32) + b1_ref[...])
    v = jnp.dot(u, w2_ref[...], preferred_element_type=jnp.float32) + b2_ref[...]
    v_scr[pl.ds(i * BM, BM), :] = v
    rows = i * BM + lax.broadcasted_iota(jnp.int32, (BM, 1), 0)
    vm = jnp.where(rows < N, v, 0.0)
    s1 = jnp.sum(vm, axis=0, keepdims=True)
    s2 = jnp.sum(vm * vm, axis=0, keepdims=True)
    upd = jnp.concatenate(
        [s1, s2, jnp.zeros((6, D_H), jnp.float32)], axis=0)

    @pl.when(i == 0)
    def _():
        st_scr[...] = jnp.zeros_like(st_scr)

    st_scr[...] += upd


def _bn(v, st, g_ref, be_ref):
    mu = st[0:1, :] * (1.0 / N)
    ex2 = st[1:2, :] * (1.0 / N)
    var = ex2 - mu * mu
    scale = g_ref[...] * lax.rsqrt(var + 1e-5)
    return jax.nn.relu(scale * (v - mu) + be_ref[...])


def _ab1_body(x_ref, p0_ref, p1_ref, w1_ref, b1_ref, w2_ref, b2_ref,
              g_ref, be_ref, o_ref, v_scr, st_scr):
    i = pl.program_id(0)

    @pl.when(i < _NB)
    def _():
        t = x_ref[...] + p0_ref[0] + p1_ref[0]
        _mlp_stats(t, w1_ref, b1_ref, w2_ref, b2_ref, v_scr, st_scr)

    @pl.when(i >= _NB)
    def _():
        v = v_scr[pl.ds((i - _NB) * BM, BM), :]
        res = _bn(v, st_scr[...], g_ref, be_ref)
        o_ref[0] = res[:, :128]
        o_ref[1] = res[:, 128:]


def _ab_body(hl_ref, hh_ref, al_ref, ah_ref, w1_ref, b1_ref, w2_ref, b2_ref,
             g_ref, be_ref, o_ref, v_scr, st_scr):
    i = pl.program_id(0)

    @pl.when(i < _NB)
    def _():
        t = jnp.concatenate([hl_ref[0] + al_ref[0], hh_ref[0] + ah_ref[0]],
                            axis=1)
        _mlp_stats(t, w1_ref, b1_ref, w2_ref, b2_ref, v_scr, st_scr)

    @pl.when(i >= _NB)
    def _():
        v = v_scr[pl.ds((i - _NB) * BM, BM), :]
        res = _bn(v, st_scr[...], g_ref, be_ref)
        o_ref[0] = res[:, :128]
        o_ref[1] = res[:, 128:]


def _ab3_body(hl_ref, hh_ref, al_ref, ah_ref, w1_ref, b1_ref, w2_ref, b2_ref,
              g_ref, be_ref, wo_ref, bo_ref, o_ref, v_scr, st_scr):
    i = pl.program_id(0)

    @pl.when(i < _NB)
    def _():
        t = jnp.concatenate([hl_ref[0] + al_ref[0], hh_ref[0] + ah_ref[0]],
                            axis=1)
        _mlp_stats(t, w1_ref, b1_ref, w2_ref, b2_ref, v_scr, st_scr)

    @pl.when(i >= _NB)
    def _():
        v = v_scr[pl.ds((i - _NB) * BM, BM), :]
        res = _bn(v, st_scr[...], g_ref, be_ref)
        o_ref[...] = jnp.dot(res, wo_ref[...],
                             preferred_element_type=jnp.float32) + bo_ref[...]


def _blk(i):
    return jnp.minimum(i, _NB - 1)


def _oblk(i):
    return jnp.maximum(i - _NB, 0)


_SCRATCH = [
    pltpu.VMEM((NP, D_H), jnp.float32),
    pltpu.VMEM((8, D_H), jnp.float32),
]

_W_SPECS = [
    pl.BlockSpec((D_H, D_H), lambda i: (0, 0)),
    pl.BlockSpec((1, D_H), lambda i: (0, 0)),
    pl.BlockSpec((D_H, D_H), lambda i: (0, 0)),
    pl.BlockSpec((1, D_H), lambda i: (0, 0)),
    pl.BlockSpec((1, D_H), lambda i: (0, 0)),
    pl.BlockSpec((1, D_H), lambda i: (0, 0)),
]


def _call_ab1(xp, a3, W1, b1, W2, b2, g, be):
    specs = [
        pl.BlockSpec((BM, 128), lambda i: (_blk(i), 0)),
        pl.BlockSpec((1, BM, 128), lambda i: (0, _blk(i), 0)),
        pl.BlockSpec((1, BM, 128), lambda i: (1, _blk(i), 0)),
        pl.BlockSpec((128, D_H), lambda i: (0, 0)),
    ] + _W_SPECS[1:]
    return pl.pallas_call(
        _ab1_body,
        grid=(2 * _NB,),
        in_specs=specs,
        out_specs=pl.BlockSpec((2, BM, 128), lambda i: (0, _oblk(i), 0)),
        out_shape=jax.ShapeDtypeStruct((2, NP, 128), jnp.float32),
        scratch_shapes=_SCRATCH,
    )(xp, a3, a3, W1, b1.reshape(1, D_H), W2, b2.reshape(1, D_H),
      g.reshape(1, D_H), be.reshape(1, D_H))


_H_SPECS = [
    pl.BlockSpec((1, BM, 128), lambda i: (0, _blk(i), 0)),
    pl.BlockSpec((1, BM, 128), lambda i: (1, _blk(i), 0)),
    pl.BlockSpec((1, BM, 128), lambda i: (0, _blk(i), 0)),
    pl.BlockSpec((1, BM, 128), lambda i: (1, _blk(i), 0)),
]


def _call_ab(h3, a3, W1, b1, W2, b2, g, be):
    return pl.pallas_call(
        _ab_body,
        grid=(2 * _NB,),
        in_specs=_H_SPECS + _W_SPECS,
        out_specs=pl.BlockSpec((2, BM, 128), lambda i: (0, _oblk(i), 0)),
        out_shape=jax.ShapeDtypeStruct((2, NP, 128), jnp.float32),
        scratch_shapes=_SCRATCH,
    )(h3, h3, a3, a3, W1, b1.reshape(1, D_H), W2, b2.reshape(1, D_H),
      g.reshape(1, D_H), be.reshape(1, D_H))


def _call_ab3(h3, a3, W1, b1, W2, b2, g, be, Wo, bo):
    specs = _H_SPECS + _W_SPECS + [
        pl.BlockSpec((D_H, D_OUT), lambda i: (0, 0)),
        pl.BlockSpec((1, D_OUT), lambda i: (0, 0)),
    ]
    return pl.pallas_call(
        _ab3_body,
        grid=(2 * _NB,),
        in_specs=specs,
        out_specs=pl.BlockSpec((BM, D_OUT), lambda i: (_oblk(i), 0)),
        out_shape=jax.ShapeDtypeStruct((NP, D_OUT), jnp.float32),
        scratch_shapes=_SCRATCH,
    )(h3, h3, a3, a3, W1, b1.reshape(1, D_H), W2, b2.reshape(1, D_H),
      g.reshape(1, D_H), be.reshape(1, D_H), Wo, bo.reshape(1, D_OUT))


def kernel(x, edge_index, W1_0, b1_0, W2_0, b2_0, g_0, be_0, W1_1, b1_1,
           W2_1, b2_1, g_1, be_1, W1_2, b1_2, W2_2, b2_2, g_2, be_2, Wo, bo):
    z128 = jnp.zeros((NP, 128), jnp.float32)
    src = edge_index[0]
    dst = edge_index[1]
    src2 = jnp.concatenate([src, src + NP])

    xp = jnp.pad(x, ((0, NP - N), (0, 0)))

    a1 = _agg_edges(xp, src, dst, z128)
    h3 = _call_ab1(xp, a1.reshape(2, NP, 128), W1_0, b1_0, W2_0, b2_0,
                   g_0, be_0)

    a2 = _agg_cols(h3.reshape(2 * NP, 128), src2, dst, z128)
    h3 = _call_ab(h3, a2.reshape(2, NP, 128), W1_1, b1_1, W2_1, b2_1,
                  g_1, be_1)

    a3 = _agg_cols(h3.reshape(2 * NP, 128), src2, dst, z128)
    out = _call_ab3(h3, a3.reshape(2, NP, 128), W1_2, b1_2, W2_2, b2_2,
                    g_2, be_2, Wo, bo)
    return out[:N]
